# Initial kernel scaffold; baseline (speedup 1.0000x reference)
import jax
import jax.numpy as jnp
from jax.experimental import pallas as pl


def kernel(hazard, is_event, event_time):
    raise NotImplementedError("write your pallas kernel here")


# speedup vs baseline: 1.3859x; 1.3859x over previous
import jax
import jax.numpy as jnp
from jax.experimental import pallas as pl
from jax.experimental.pallas import tpu as pltpu

_N = 4096
_NB = 8
_BR = _N // _NB
_EPS = 1e-07


def _tc_body(et_col, et_row, h_col, h_row, ev_col, out_ref, acc):
    i = pl.program_id(0)

    @pl.when(i == 0)
    def _init():
        acc[0] = 0.0
        acc[1] = 0.0

    mask = (et_row[...] >= et_col[...]).astype(jnp.float32)
    e_row = jnp.exp(h_row[...])
    s = jnp.sum(mask * e_row, axis=1, keepdims=True)
    ev = ev_col[...]
    acc[0] += jnp.sum(ev * (jnp.log(s) - h_col[...]))
    acc[1] += jnp.sum(ev)

    @pl.when(i == _NB - 1)
    def _fin():
        out_ref[0, 0] = acc[0] / (acc[1] + _EPS)


def kernel(hazard, is_event, event_time):
    h = hazard.reshape(-1).astype(jnp.float32)
    ev = is_event.astype(jnp.float32).reshape(-1)
    et = event_time.astype(jnp.float32)

    col = lambda x: x.reshape(_N, 1)
    row = lambda x: x.reshape(1, _N)

    out = pl.pallas_call(
        _tc_body,
        grid=(_NB,),
        in_specs=[
            pl.BlockSpec((_BR, 1), lambda i: (i, 0)),
            pl.BlockSpec((1, _N), lambda i: (0, 0)),
            pl.BlockSpec((_BR, 1), lambda i: (i, 0)),
            pl.BlockSpec((1, _N), lambda i: (0, 0)),
            pl.BlockSpec((_BR, 1), lambda i: (i, 0)),
        ],
        out_specs=pl.BlockSpec(memory_space=pltpu.SMEM),
        out_shape=jax.ShapeDtypeStruct((1, 1), jnp.float32),
        scratch_shapes=[pltpu.SMEM((2,), jnp.float32)],
        compiler_params=pltpu.CompilerParams(
            dimension_semantics=("arbitrary",)),
    )(col(et), row(et), col(h), row(h), col(ev))
    return out[0, 0]
